# baseline (speedup 1.0000x reference)
import functools

import jax
import jax.numpy as jnp
from jax import lax
from jax.experimental import pallas as pl
from jax.experimental.pallas import tpu as pltpu
from jax.experimental.pallas import tpu_sc as plsc

B, T, EMB = 4096, 200, 128
MAX_LEN = 8192
N = B * T
NC, NS = 2, 16
NW = NC * NS
PER_W = N // NW
CHUNK = 80
NCHUNK = PER_W // CHUNK

NBUF = 5
NIDX = 8
PERIOD = 40
HEAD = 8
TAIL = 32


@functools.partial(
    pl.kernel,
    out_type=jax.ShapeDtypeStruct((N, EMB), jnp.float32),
    mesh=plsc.VectorSubcoreMesh(core_axis_name="c", subcore_axis_name="s"),
    scratch_types=[
        pltpu.VMEM((NIDX, CHUNK), jnp.int32),
        pltpu.VMEM_SHARED((MAX_LEN, EMB), jnp.float32),
    ]
    + [pltpu.VMEM((CHUNK, EMB), jnp.float32) for _ in range(NBUF)]
    + [pltpu.SemaphoreType.DMA for _ in range(2 * NBUF + NIDX)],
)
def _sc_gather(table, idx, out, idx_v, table_sh, *bufs):
    rows = bufs[:NBUF]
    gsems = bufs[NBUF:2 * NBUF]
    osems = bufs[2 * NBUF:3 * NBUF]
    isems = bufs[3 * NBUF:]
    sid = lax.axis_index("s")
    w = sid * NC + lax.axis_index("c")
    base = w * PER_W

    stripe = MAX_LEN // NS
    pltpu.sync_copy(
        table.at[pl.ds(sid * stripe, stripe)],
        table_sh.at[pl.ds(sid * stripe, stripe)],
    )
    plsc.subcore_barrier()

    def i_start(g, s):
        pltpu.async_copy(idx.at[w, g], idx_v.at[s], isems[s])

    def i_wait(g, s):
        pltpu.make_async_copy(idx.at[w, g], idx_v.at[s], isems[s]).wait()

    def g_start(b, s):
        pltpu.async_copy(table_sh.at[idx_v.at[s]], rows[b], gsems[b])

    def g_wait(b, s):
        pltpu.make_async_copy(table_sh.at[idx_v.at[s]], rows[b], gsems[b]).wait()

    def o_start(g, b):
        pltpu.async_copy(rows[b], out.at[pl.ds(base + g * CHUNK, CHUNK)], osems[b])

    def o_wait(g, b):
        pltpu.make_async_copy(
            rows[b], out.at[pl.ds(base + g * CHUNK, CHUNK)], osems[b]
        ).wait()

    def chunk_ops(g, sb, s8, do_owait=True, do_istart=True, do_gstart=True):
        if do_istart:
            i_start(g + 4, (s8 + 4) % NIDX)
        if do_owait:
            o_wait(g - 3, (sb + 2) % NBUF)
        if do_gstart:
            i_wait(g + 2, (s8 + 2) % NIDX)
            g_start((sb + 2) % NBUF, (s8 + 2) % NIDX)
        g_wait(sb, s8)
        o_start(g, sb)

    for g in range(4):
        i_start(g, g)
    for g in range(2):
        i_wait(g, g)
        g_start(g, g)
    for g in range(HEAD):
        chunk_ops(g, g % NBUF, g % NIDX, do_owait=g >= 3)

    def body(t, carry):
        for j in range(PERIOD):
            g = HEAD + t * PERIOD + j
            chunk_ops(g, (HEAD + j) % NBUF, (HEAD + j) % NIDX)
        return carry

    lax.fori_loop(0, (NCHUNK - HEAD - TAIL) // PERIOD, body, 0)

    for j in range(TAIL):
        g = NCHUNK - TAIL + j
        chunk_ops(
            g,
            g % NBUF,
            g % NIDX,
            do_istart=g + 4 < NCHUNK,
            do_gstart=g + 2 < NCHUNK,
        )
    for g in range(NCHUNK - 3, NCHUNK):
        o_wait(g, g % NBUF)


def kernel(encoding, x):
    idx = x.reshape(N).astype(jnp.int32).reshape(NW, NCHUNK, CHUNK)
    out = _sc_gather(encoding, idx)
    return out.reshape(B, T, EMB)


# speedup vs baseline: 1.0054x; 1.0054x over previous
import functools

import jax
import jax.numpy as jnp
from jax import lax
from jax.experimental import pallas as pl
from jax.experimental.pallas import tpu as pltpu
from jax.experimental.pallas import tpu_sc as plsc

B, T, EMB = 4096, 200, 128
MAX_LEN = 8192
N = B * T
NC, NS = 2, 16
NW = NC * NS
PER_W = N // NW
CHUNK = 80
NCHUNK = PER_W // CHUNK

NBUF = 4
NIDX = 8


@functools.partial(
    pl.kernel,
    out_type=jax.ShapeDtypeStruct((N, EMB), jnp.float32),
    mesh=plsc.VectorSubcoreMesh(core_axis_name="c", subcore_axis_name="s"),
    scratch_types=[
        pltpu.VMEM((NIDX, CHUNK), jnp.int32),
        pltpu.VMEM_SHARED((MAX_LEN, EMB), jnp.float32),
    ]
    + [pltpu.VMEM((CHUNK, EMB), jnp.float32) for _ in range(NBUF)]
    + [pltpu.SemaphoreType.DMA for _ in range(2 * NBUF + NIDX)],
)
def _sc_gather(table, idx, out, idx_v, table_sh, *bufs):
    rows = bufs[:NBUF]
    gsems = bufs[NBUF:2 * NBUF]
    osems = bufs[2 * NBUF:3 * NBUF]
    isems = bufs[3 * NBUF:]
    sid = lax.axis_index("s")
    w = sid * NC + lax.axis_index("c")
    base = w * PER_W

    stripe = MAX_LEN // NS
    pltpu.sync_copy(
        table.at[pl.ds(sid * stripe, stripe)],
        table_sh.at[pl.ds(sid * stripe, stripe)],
    )
    plsc.subcore_barrier()

    def i_start(g, s):
        pltpu.async_copy(idx.at[w, g], idx_v.at[s], isems[s])

    def i_wait(g, s):
        pltpu.make_async_copy(idx.at[w, g], idx_v.at[s], isems[s]).wait()

    def g_start(b, s):
        pltpu.async_copy(table_sh.at[idx_v.at[s]], rows[b], gsems[b])

    def g_wait(b, s):
        pltpu.make_async_copy(table_sh.at[idx_v.at[s]], rows[b], gsems[b]).wait()

    def o_start(g, b):
        pltpu.async_copy(rows[b], out.at[pl.ds(base + g * CHUNK, CHUNK)], osems[b])

    def o_wait(g, b):
        pltpu.make_async_copy(
            rows[b], out.at[pl.ds(base + g * CHUNK, CHUNK)], osems[b]
        ).wait()

    def chunk_ops(g, s, do_owait=True, do_istart=True, do_gstart=True):
        b = s % NBUF
        if do_istart:
            i_start(g + NBUF, (s + NBUF) % NIDX)
        if do_owait:
            o_wait(g - 3, (b + 1) % NBUF)
        if do_gstart:
            i_wait(g + 1, (s + 1) % NIDX)
            g_start((b + 1) % NBUF, (s + 1) % NIDX)
        g_wait(b, s)
        o_start(g, b)

    for g in range(NBUF):
        i_start(g, g)
    i_wait(0, 0)
    g_start(0, 0)
    for g in range(NIDX):
        chunk_ops(g, g, do_owait=g >= 3)

    def body(t, carry):
        for j in range(NIDX):
            chunk_ops(t * NIDX + j, j)
        return carry

    lax.fori_loop(1, NCHUNK // NIDX - 1, body, 0)

    for j in range(NIDX):
        g = NCHUNK - NIDX + j
        chunk_ops(
            g,
            j,
            do_istart=g + NBUF < NCHUNK,
            do_gstart=g + 1 < NCHUNK,
        )
    for g in range(NCHUNK - 3, NCHUNK):
        o_wait(g, g % NBUF)


def kernel(encoding, x):
    idx = x.reshape(N).astype(jnp.int32).reshape(NW, NCHUNK, CHUNK)
    out = _sc_gather(encoding, idx)
    return out.reshape(B, T, EMB)
